# Initial kernel scaffold; baseline (speedup 1.0000x reference)
import jax
import jax.numpy as jnp
from jax.experimental import pallas as pl


def kernel(eq1_node_idx, eq1_edge_index, eq1_var_idx, tar_node_idx, tar_edge_index, operation, labels, symbol, Wo, ov, W_lin, b_lin, Wl1, bl1, Wr1, Wl2, bl2, Wr2):
    raise NotImplementedError("write your pallas kernel here")


# speedup vs baseline: 16.9515x; 16.9515x over previous
import functools

import jax
import jax.numpy as jnp
from jax import lax
from jax.experimental import pallas as pl
from jax.experimental.pallas import tpu as pltpu
from jax.experimental.pallas import tpu_sc as plsc

N_NODES = 10000
D = 768
HK = 16
NSUB = 16
CW = 128
F32 = jnp.float32
I32 = jnp.int32


def _split3(a):
    a0 = a.astype(jnp.bfloat16).astype(F32)
    r = a - a0
    a1 = r.astype(jnp.bfloat16).astype(F32)
    return a0, a1, r - a1


def _dot(a, b):
    a0, a1, a2 = _split3(a)
    b0, b1, b2 = _split3(b)
    d = lambda x, y: jnp.dot(x, y, preferred_element_type=F32)
    return (((d(a0, b2) + d(a2, b0)) + d(a1, b1))
            + (d(a0, b1) + d(a1, b0))) + d(a0, b0)


def _dg3(a, b, dn):
    a0, a1, a2 = _split3(a)
    b0, b1, b2 = _split3(b)
    d = lambda x, y: lax.dot_general(x, y, dn, preferred_element_type=F32)
    return (((d(a0, b2) + d(a2, b0)) + d(a1, b1))
            + (d(a0, b1) + d(a1, b0))) + d(a0, b0)


def _sc_body(nchunks, hsz, csz, npad,
             edges, nodeidx, varv, zeros,
             hist_out, cnt_out, w_out, cvar_out,
             src_v, dst_v, key_c, dst_c, src_c, ones_c, wv_c, cvv_c,
             sym_v, cnt_l, var_v,
             hist_sp, cnt_sp, w_sp, cvar_sp):
    c = lax.axis_index("c")
    s = lax.axis_index("s")
    hslc = hsz // NSUB
    cslc = csz // NSUB

    pltpu.sync_copy(edges.at[c, 0, pl.ds(s * nchunks, nchunks)], src_v)
    pltpu.sync_copy(edges.at[c, 1, pl.ds(s * nchunks, nchunks)], dst_v)
    pltpu.sync_copy(nodeidx.at[c, 0], sym_v)
    pltpu.sync_copy(varv, var_v)

    pltpu.sync_copy(zeros.at[pl.ds(s * hslc, hslc)],
                    hist_sp.at[pl.ds(s * hslc, hslc)])
    pltpu.sync_copy(zeros.at[pl.ds(s * cslc, cslc)],
                    cnt_sp.at[pl.ds(s * cslc, cslc)])
    pltpu.sync_copy(zeros.at[pl.ds(s * cslc, cslc)],
                    w_sp.at[pl.ds(s * cslc, cslc)])
    pltpu.sync_copy(zeros.at[pl.ds(s * cslc, cslc)],
                    cvar_sp.at[pl.ds(s * cslc, cslc)])
    plsc.subcore_barrier()

    one16 = jnp.full((16,), 1.0, F32)
    for k in range(CW // 16):
        ones_c[pl.ds(k * 16, 16)] = one16

    def a_chunk(j, carry):
        for k in range(CW // 16):
            sl = pl.ds(k * 16, 16)
            src = src_v[j, sl]
            dst = dst_v[j, sl]
            sym = plsc.load_gather(sym_v, [src])
            key_c[sl] = dst * HK + sym
            dst_c[sl] = dst
        pltpu.sync_copy(ones_c, hist_sp.at[key_c], add=True)
        pltpu.sync_copy(ones_c, cnt_sp.at[dst_c], add=True)
        return carry

    lax.fori_loop(0, nchunks, a_chunk, 0)
    plsc.subcore_barrier()

    pltpu.sync_copy(cnt_sp, cnt_l)
    var16 = var_v[...]

    def b_chunk(j, carry):
        for k in range(CW // 16):
            sl = pl.ds(k * 16, 16)
            src = src_v[j, sl]
            dst = dst_v[j, sl]
            cv = plsc.load_gather(cnt_l, [dst])
            cm = jnp.maximum(cv, 1.0)
            r = 1.0 / cm
            r = r * (2.0 - cm * r)
            wv_c[sl] = r * (2.0 - cm * r)
            cvv_c[sl] = jnp.where(dst == var16, one16,
                                  jnp.zeros((16,), F32))
            src_c[sl] = src
        pltpu.sync_copy(wv_c, w_sp.at[src_c], add=True)
        pltpu.sync_copy(cvv_c, cvar_sp.at[src_c], add=True)
        return carry

    lax.fori_loop(0, nchunks, b_chunk, 0)
    plsc.subcore_barrier()

    pltpu.sync_copy(hist_sp.at[pl.ds(s * hslc, hslc)],
                    hist_out.at[c, 0, pl.ds(s * hslc, hslc)])
    pltpu.sync_copy(cnt_sp.at[pl.ds(s * cslc, cslc)],
                    cnt_out.at[c, 0, pl.ds(s * cslc, cslc)])
    pltpu.sync_copy(w_sp.at[pl.ds(s * cslc, cslc)],
                    w_out.at[c, 0, pl.ds(s * cslc, cslc)])
    pltpu.sync_copy(cvar_sp.at[pl.ds(s * cslc, cslc)],
                    cvar_out.at[c, 0, pl.ds(s * cslc, cslc)])


def _edge_stats(edges, nodeidx, varv, zeros, nchunks, hsz, csz, npad):
    ept = nchunks * CW
    mesh = plsc.VectorSubcoreMesh(core_axis_name="c", subcore_axis_name="s")
    body = functools.partial(_sc_body, nchunks, hsz, csz, npad)
    k = pl.kernel(
        body,
        out_type=[
            jax.ShapeDtypeStruct((2, 1, hsz), F32),
            jax.ShapeDtypeStruct((2, 1, csz), F32),
            jax.ShapeDtypeStruct((2, 1, csz), F32),
            jax.ShapeDtypeStruct((2, 1, csz), F32),
        ],
        mesh=mesh,
        compiler_params=pltpu.CompilerParams(needs_layout_passes=False),
        scratch_types=[
            pltpu.VMEM((nchunks, CW), I32),
            pltpu.VMEM((nchunks, CW), I32),
            pltpu.VMEM((CW,), I32),
            pltpu.VMEM((CW,), I32),
            pltpu.VMEM((CW,), I32),
            pltpu.VMEM((CW,), F32),
            pltpu.VMEM((CW,), F32),
            pltpu.VMEM((CW,), F32),
            pltpu.VMEM((npad,), I32),
            pltpu.VMEM((csz,), F32),
            pltpu.VMEM((16,), I32),
            pltpu.VMEM_SHARED((hsz,), F32),
            pltpu.VMEM_SHARED((csz,), F32),
            pltpu.VMEM_SHARED((csz,), F32),
            pltpu.VMEM_SHARED((csz,), F32),
        ],
    )
    return k(edges, nodeidx, varv, zeros)


def _tc_body(ntiles, tsz,
             hist_ref, nidx_ref, cnt_ref, wt_ref, sym_ref,
             wl1t_ref, wr1t_ref, wl2t_ref, wr2t_ref,
             bl1_ref, bl2_ref, wlint_ref, blin_ref,
             wo_ref, ov_ref, op_ref, lab_ref, cv_ref,
             loss_ref, sc_ref, emb_ref,
             acc, ab):
    i = pl.program_id(0)
    g = i // ntiles

    @pl.when(i == 0)
    def _init():
        s16 = sym_ref[...]
        ab[0:16, :] = _dot(s16, wl1t_ref[...])
        ab[16:32, :] = _dot(s16, wr1t_ref[...])
        acc[...] = jnp.zeros((2, 8, D), F32)

    cnt = cnt_ref[0, 0, :]
    c = jnp.maximum(cnt, 1.0)
    recip = 1.0 / c
    recip = recip * (2.0 - c * recip)
    m = hist_ref[0] * recip[:, None]
    ni = nidx_ref[0, 0, :]
    onehot = (ni[:, None] == lax.broadcasted_iota(I32, (tsz, HK), 1)
              ).astype(F32)
    x = jnp.concatenate([m, onehot], axis=1)
    h = jnp.maximum(_dot(x, ab[...]) + bl1_ref[...], 0.0)
    wt = wt_ref[0]
    contrib = _dg3(wt, h, (((0,), (0,)), ((), ())))
    cur = acc[pl.ds(g, 1)]
    acc[pl.ds(g, 1)] = cur + contrib[None]

    @pl.when(i == 2 * ntiles - 1)
    def _epilogue():
        s1 = acc[0]
        s2 = acc[1]
        n = float(N_NODES)
        u1, v1, rc1, re1 = s1[0:1], s1[1:2], s1[2:3], s1[3:4]
        u2, v2 = s2[0:1], s2[1:2]
        cv = cv_ref[...]
        wl2t = wl2t_ref[...]
        wr2t = wr2t_ref[...]
        bl2 = bl2_ref[...]
        rcv = 1.0 / cv
        rcv = rcv * (2.0 - cv * rcv)
        e1 = _dot(u1 * (1.0 / n), wl2t) + bl2 + _dot(v1 * (1.0 / n), wr2t)
        e2 = _dot(rc1 * rcv, wl2t) + bl2 + _dot(re1, wr2t)
        et = _dot(u2 * (1.0 / n), wl2t) + bl2 + _dot(v2 * (1.0 / n), wr2t)
        eo = (_dot(e1, wlint_ref[0]) + _dot(e2, wlint_ref[1])
              + _dot(e1 * e2, wlint_ref[2]) + blin_ref[...])
        op1h = op_ref[...]
        wo_sel = _dot(op1h, wo_ref[...])
        ov_sel = _dot(op1h, ov_ref[...])
        a = eo * wo_sel
        b = et + ov_sel
        na = jnp.maximum(jnp.sqrt(jnp.sum(a * a, axis=1, keepdims=True)),
                         1e-8)
        nb = jnp.maximum(jnp.sqrt(jnp.sum(b * b, axis=1, keepdims=True)),
                         1e-8)
        q = na * nb
        rq = 1.0 / q
        rq = rq * (2.0 - q * rq)
        sc = jnp.sum(a * b, axis=1, keepdims=True) * rq
        lab = lab_ref[...]
        loss_ref[...] = (sc - lab) ** 2
        sc_ref[...] = sc
        emb_ref[...] = a


def _dense(histr, nidx, cntb, wt, sym16, wl1t, wr1t, wl2t, wr2t,
           bl1, bl2, wlint, blin, wo, ov, op1h, lab, cv, ntiles, tsz):
    full = lambda shp: pl.BlockSpec(shp, lambda i: tuple(0 for _ in shp))
    grid = (2 * ntiles,)
    body = functools.partial(_tc_body, ntiles, tsz)
    return pl.pallas_call(
        body,
        grid=grid,
        in_specs=[
            pl.BlockSpec((1, tsz, HK), lambda i: (i, 0, 0)),
            pl.BlockSpec((1, 1, tsz), lambda i: (i, 0, 0)),
            pl.BlockSpec((1, 1, tsz), lambda i: (i, 0, 0)),
            pl.BlockSpec((1, tsz, 8), lambda i: (i, 0, 0)),
            full((16, D)),
            full((D, D)), full((D, D)), full((D, D)), full((D, D)),
            full((1, D)), full((1, D)),
            full((3, D, D)), full((1, D)),
            full((16, D)), full((16, D)), full((1, 16)),
            full((1, 1)), full((1, 1)),
        ],
        out_specs=[full((1, 1)), full((1, 1)), full((1, D))],
        out_shape=[
            jax.ShapeDtypeStruct((1, 1), F32),
            jax.ShapeDtypeStruct((1, 1), F32),
            jax.ShapeDtypeStruct((1, D), F32),
        ],
        scratch_shapes=[
            pltpu.VMEM((2, 8, D), F32),
            pltpu.VMEM((32, D), F32),
        ],
    )(histr, nidx, cntb, wt, sym16, wl1t, wr1t, wl2t, wr2t,
      bl1, bl2, wlint, blin, wo, ov, op1h, lab, cv)


def kernel(eq1_node_idx, eq1_edge_index, eq1_var_idx, tar_node_idx,
           tar_edge_index, operation, labels, symbol, Wo, ov, W_lin, b_lin,
           Wl1, bl1, Wr1, Wl2, bl2, Wr2):
    n = N_NODES
    e = eq1_edge_index.shape[1]

    nchunks = -(-e // (NSUB * CW * 8)) * 8
    e_pad = NSUB * CW * nchunks
    csz = -(-(n + 1) // (NSUB * CW)) * (NSUB * CW)
    hsz = csz * HK
    npad = csz

    def pad_edges(ei):
        ei = ei.astype(I32)
        ei = jnp.pad(ei, ((0, 0), (0, e_pad - e)), constant_values=n)
        return ei.reshape(2, NSUB * nchunks, CW)

    edges = jnp.stack([pad_edges(eq1_edge_index), pad_edges(tar_edge_index)])
    nodeidx = jnp.stack([
        jnp.pad(eq1_node_idx.astype(I32), (0, npad - n)),
        jnp.pad(tar_node_idx.astype(I32), (0, npad - n)),
    ]).reshape(2, 1, npad)
    var = jnp.asarray(eq1_var_idx, I32)
    varv = jnp.full((16,), var, I32)
    zeros = jnp.zeros((hsz,), F32)

    hist2, cnt2, w2, cvar2 = _edge_stats(edges, nodeidx, varv, zeros,
                                         nchunks, hsz, csz, npad)

    ntiles = 10
    tsz = n // ntiles
    histr = hist2.reshape(2, csz, HK)[:, :n, :].reshape(2 * ntiles, tsz, HK)
    cnt = cnt2.reshape(2, csz)[:, :n]
    cntb = cnt.reshape(2 * ntiles, 1, tsz)
    nidx = nodeidx.reshape(2, npad)[:, :n].reshape(2 * ntiles, 1, tsz)
    w = w2.reshape(2, csz)[:, :n]
    cvar = cvar2.reshape(2, csz)[:, :n]
    evar = (jnp.arange(n, dtype=I32) == var).astype(F32)
    zcol = jnp.zeros((2, n), F32)
    wt = jnp.stack([
        w,
        jnp.ones((2, n), F32),
        jnp.stack([cvar[0], jnp.zeros((n,), F32)]),
        jnp.stack([evar, jnp.zeros((n,), F32)]),
        zcol, zcol, zcol, zcol,
    ], axis=2).reshape(2 * ntiles, tsz, 8)

    sym16 = jnp.zeros((16, D), F32).at[:symbol.shape[0]].set(symbol)
    wlint = W_lin.reshape(D, 3, D).transpose(1, 2, 0)
    op1h = (jnp.arange(16, dtype=I32) ==
            (jnp.asarray(operation, I32) - 1)).astype(F32).reshape(1, 16)
    cv = jnp.maximum(cnt[0, var], 1.0).reshape(1, 1)
    lab = labels.astype(F32).reshape(1, 1)

    loss, scores, emb = _dense(
        histr, nidx, cntb, wt, sym16,
        Wl1.T, Wr1.T, Wl2.T, Wr2.T,
        bl1.reshape(1, D), bl2.reshape(1, D), wlint, b_lin.reshape(1, D),
        Wo, ov, op1h, lab, cv, ntiles, tsz)

    return (loss.reshape(()), scores.reshape(1), labels, emb.reshape(D))
